# baseline (speedup 1.0000x reference)
import jax
import jax.numpy as jnp
from jax.experimental import pallas as pl
from jax.experimental.pallas import tpu as pltpu

_B, _S, _D = 4, 1024, 256
_H, _HD = 4, 64
_HID = 128
_C = 64
_NC = _S // _C
_MAXN = 10.0
_BG = 2
_G = _BG * _H

_HI = jax.lax.Precision.HIGHEST


def _sp(x):
    hi = x.astype(jnp.bfloat16)
    lo = (x - hi.astype(jnp.float32)).astype(jnp.bfloat16)
    return hi, lo


def _dot3(a, b, dn=None):
    ah, al = a
    bh, bl = b
    if dn is None:
        f = lambda u, v: jnp.dot(u, v, preferred_element_type=jnp.float32)
    else:
        f = lambda u, v: jax.lax.dot_general(
            u, v, dn, preferred_element_type=jnp.float32)
    return f(ah, bh) + (f(ah, bl) + f(al, bh))


def _logsig(z):
    return -(jnp.maximum(-z, 0.0) + jnp.log(1.0 + jnp.exp(-jnp.abs(z))))


def _silu(z):
    return z * jax.nn.sigmoid(z)


def _dsilu(z):
    s = jax.nn.sigmoid(z)
    return s * (1.0 + z * (1.0 - s))


def _body(x_ref, wkT, wvT, wqT, wgT, w1T, w2T, woutT, out_ref,
          ks, vs, qs, rs, cf):
    wk = wkT[...]
    wv = wvT[...]
    wq = wqT[...]
    for bb in range(_BG):
        for r in range(4):
            sl = slice(r * 256, (r + 1) * 256)
            xb = x_ref[bb, sl, :]
            ks[bb, sl, :] = jnp.dot(xb, wk, preferred_element_type=jnp.float32)
            vs[bb, sl, :] = jnp.dot(xb, wv, preferred_element_type=jnp.float32)
            qs[bb, sl, :] = jnp.dot(xb, wq, preferred_element_type=jnp.float32)

    row = jax.lax.broadcasted_iota(jnp.int32, (_C, _C), 0)
    col = jax.lax.broadcasted_iota(jnp.int32, (_C, _C), 1)
    lmat = jnp.where(col <= row, 1.0, 0.0).astype(jnp.float32)
    mask_ge = col >= row

    for bb in range(_BG):
        for c in range(_NC):
            rows = slice(c * _C, (c + 1) * _C)
            z = jnp.dot(x_ref[bb, rows, :], wgT[...],
                        preferred_element_type=jnp.float32)
            sig = jax.nn.sigmoid(z)
            lsp = _logsig(z)
            lsn = _logsig(-z)
            ll = jnp.concatenate([lsp[:, _H:2 * _H], lsn[:, 2 * _H:3 * _H]],
                                 axis=1)
            cum = jnp.dot(lmat, ll, precision=_HI,
                          preferred_element_type=jnp.float32)
            tot = cum[_C - 1:_C, :]
            le = cum[:, :_H]
            la = tot[:, :_H] - le
            lb = tot[:, _H:] - cum[:, _H:]
            lbe = lb + le
            a_row = jnp.exp(tot[:, :_H])
            ca_row = jnp.exp(tot[:, _H:])
            sme_row = jnp.sum(jnp.exp(lbe), axis=0, keepdims=True)
            ela = jnp.exp(la)
            lbe_t = lbe.T

            ccols = []
            k2s = []
            for h in range(_H):
                diff = lbe_t[h:h + 1, :] - le[:, h:h + 1]
                cmat = jnp.exp(jnp.where(mask_ge, diff, -1e30))
                ccols.append(jnp.sum(cmat, axis=1, keepdims=True) *
                             sig[:, h:h + 1])
                fsl = slice(h * _HD, (h + 1) * _HD)
                kc = ks[bb, rows, fsl]
                kn = jnp.sqrt(jnp.sum(kc * kc, axis=1, keepdims=True))
                kinv = 1.0 / jnp.maximum(kn, 1e-12)
                ks[bb, rows, fsl] = kc * kinv
                k2s.append(jnp.square(kn * kinv))
                qc = qs[bb, rows, fsl]
                qn = jnp.sqrt(jnp.sum(qc * qc, axis=1, keepdims=True))
                qs[bb, rows, fsl] = qc / jnp.maximum(qn, 1e-12)

            blk = jnp.concatenate(
                [ela * sig[:, :_H]] + ccols + k2s +
                [jnp.broadcast_to(a_row, (_C, _H)),
                 jnp.broadcast_to(ca_row, (_C, _H)),
                 jnp.broadcast_to(sme_row, (_C, _H))], axis=1)
            cf[bb, rows, :] = blk

    def chunk(c, carry):
        t0 = c * _C
        new_carry = list(carry)
        for bb in range(_BG):
            cfc = cf[bb, pl.ds(t0, _C), :]
            for h in range(_H):
                p = bb * _H + h
                p1t, p2t, m1t, m2t = new_carry[p]
                fsl = slice(h * _HD, (h + 1) * _HD)
                kc = ks[bb, pl.ds(t0, _C), fsl]
                vc = vs[bb, pl.ds(t0, _C), fsl]
                qc = qs[bb, pl.ds(t0, _C), fsl]

                qk = jnp.concatenate([qc, kc], axis=0)
                pre2 = jnp.dot(qk, p1t,
                               preferred_element_type=jnp.float32)
                act = _silu(pre2)
                mid2 = jnp.dot(act, p2t,
                               preferred_element_type=jnp.float32)
                outq = qc + mid2[:_C, :]
                rs[p, pl.ds(t0, _C), :] = outq

                pre = pre2[_C:, :]
                h1 = act[_C:, :]
                y = kc + mid2[_C:, :]
                e = 2.0 * (y - vc)
                dsv = jax.lax.dot_general(
                    e, p2t, (((1,), (1,)), ((), ())),
                    preferred_element_type=jnp.float32) * _dsilu(pre)

                e2 = jnp.sum(e * e, axis=1, keepdims=True)
                h12 = jnp.sum(h1 * h1, axis=1, keepdims=True)
                d2 = jnp.sum(dsv * dsv, axis=1, keepdims=True)
                k2 = cfc[:, 2 * _H + h:2 * _H + h + 1]
                s2 = 1.0 / jnp.maximum(jnp.sqrt(e2 * h12) / _MAXN, 1.0)
                s1 = 1.0 / jnp.maximum(jnp.sqrt(d2 * k2) / _MAXN, 1.0)

                wm1 = cfc[:, h:h + 1] * s1
                wp1 = cfc[:, _H + h:_H + h + 1] * s1
                wm2 = cfc[:, h:h + 1] * s2
                wp2 = cfc[:, _H + h:_H + h + 1] * s2

                dn_ta = (((0,), (0,)), ((), ()))
                dup1 = _dot3(
                    _sp(kc),
                    _sp(jnp.concatenate([dsv * wm1, dsv * wp1], axis=1)),
                    dn_ta)
                dup2 = _dot3(
                    _sp(h1),
                    _sp(jnp.concatenate([e * wm2, e * wp2], axis=1)),
                    dn_ta)
                dm1, dp1 = dup1[:, :_HID], dup1[:, _HID:]
                dm2, dp2 = dup2[:, :_HD], dup2[:, _HD:]

                a_s = cfc[0:1, 3 * _H + h:3 * _H + h + 1]
                ca_s = cfc[0:1, 4 * _H + h:4 * _H + h + 1]
                sm_s = cfc[0:1, 5 * _H + h:5 * _H + h + 1]
                new_carry[p] = (ca_s * p1t + sm_s * m1t - dp1,
                                ca_s * p2t + sm_s * m2t - dp2,
                                a_s * m1t - dm1,
                                a_s * m2t - dm2)
        return tuple(new_carry)

    init = tuple((w1T[h], w2T[h],
                  jnp.zeros((_HD, _HID), jnp.float32),
                  jnp.zeros((_HID, _HD), jnp.float32))
                 for bb in range(_BG) for h in range(_H))

    def chunk2(i, carry):
        return chunk(2 * i + 1, chunk(2 * i, carry))

    jax.lax.fori_loop(0, _NC // 2, chunk2, init)

    for bb in range(_BG):
        for r in range(4):
            sl = slice(r * 256, (r + 1) * 256)
            acc = jnp.dot(rs[bb * _H, sl, :], woutT[0:_HD, :],
                          preferred_element_type=jnp.float32)
            for h in range(1, _H):
                acc = acc + jnp.dot(rs[bb * _H + h, sl, :],
                                    woutT[h * _HD:(h + 1) * _HD, :],
                                    preferred_element_type=jnp.float32)
            out_ref[bb, sl, :] = acc


def _call(x, wkT, wvT, wqT, wgT, w1T, w2T, woutT, interpret=False):
    return pl.pallas_call(
        _body,
        out_shape=jax.ShapeDtypeStruct((_B, _S, _D), jnp.float32),
        grid=(_B // _BG,),
        in_specs=[
            pl.BlockSpec((_BG, _S, _D), lambda i: (i, 0, 0)),
            pl.BlockSpec((_D, _D), lambda i: (0, 0)),
            pl.BlockSpec((_D, _D), lambda i: (0, 0)),
            pl.BlockSpec((_D, _D), lambda i: (0, 0)),
            pl.BlockSpec((_D, 3 * _H), lambda i: (0, 0)),
            pl.BlockSpec((_H, _HD, _HID), lambda i: (0, 0, 0)),
            pl.BlockSpec((_H, _HID, _HD), lambda i: (0, 0, 0)),
            pl.BlockSpec((_D, _D), lambda i: (0, 0)),
        ],
        out_specs=pl.BlockSpec((_BG, _S, _D), lambda i: (i, 0, 0)),
        scratch_shapes=[
            pltpu.VMEM((_BG, _S, _D), jnp.float32),
            pltpu.VMEM((_BG, _S, _D), jnp.float32),
            pltpu.VMEM((_BG, _S, _D), jnp.float32),
            pltpu.VMEM((_G, _S, _HD), jnp.float32),
            pltpu.VMEM((_BG, _S, 6 * _H), jnp.float32),
        ],
        compiler_params=pltpu.CompilerParams(
            dimension_semantics=("parallel",),
        ),
        name="neural_memory",
        interpret=interpret,
    )(x, wkT, wvT, wqT, wgT, w1T, w2T, woutT)


def kernel(x, Wk, Wv, Wq, Wlr, Wmom, Wdec, W1, W2, Wout, interpret=False):
    wgT = jnp.concatenate([Wlr, Wmom, Wdec], axis=0).T
    return _call(x, Wk.T, Wv.T, Wq.T, wgT,
                 jnp.swapaxes(W1, 1, 2), jnp.swapaxes(W2, 1, 2), Wout.T,
                 interpret=interpret)


# speedup vs baseline: 1.2338x; 1.2338x over previous
import jax
import jax.numpy as jnp
from jax.experimental import pallas as pl
from jax.experimental.pallas import tpu as pltpu

_B, _S, _D = 4, 1024, 256
_H, _HD = 4, 64
_HID = 128
_C = 64
_NC = _S // _C
_MAXN = 10.0
_BG = 2
_G = _BG * _H

_HI = jax.lax.Precision.HIGHEST


def _sp(x):
    hi = x.astype(jnp.bfloat16)
    lo = (x - hi.astype(jnp.float32)).astype(jnp.bfloat16)
    return hi, lo


def _dot3(a, b, dn=None):
    ah, al = a
    bh, bl = b
    if dn is None:
        f = lambda u, v: jnp.dot(u, v, preferred_element_type=jnp.float32)
    else:
        f = lambda u, v: jax.lax.dot_general(
            u, v, dn, preferred_element_type=jnp.float32)
    return f(ah, bh) + (f(ah, bl) + f(al, bh))


def _logsig(z):
    return -(jnp.maximum(-z, 0.0) + jnp.log(1.0 + jnp.exp(-jnp.abs(z))))


def _silu(z):
    return z * jax.nn.sigmoid(z)


def _dsilu(z):
    s = jax.nn.sigmoid(z)
    return s * (1.0 + z * (1.0 - s))


def _body(x_ref, wkT, wvT, wqT, wgT, w1T, w2T, woutT, out_ref, ks, vs, qs, rs):
    wk = wkT[...]
    wv = wvT[...]
    wq = wqT[...]
    for bb in range(_BG):
        for r in range(4):
            sl = slice(r * 256, (r + 1) * 256)
            xb = x_ref[bb, sl, :]
            ks[bb, sl, :] = jnp.dot(xb, wk, preferred_element_type=jnp.float32)
            vs[bb, sl, :] = jnp.dot(xb, wv, preferred_element_type=jnp.float32)
            qs[bb, sl, :] = jnp.dot(xb, wq, preferred_element_type=jnp.float32)

    row = jax.lax.broadcasted_iota(jnp.int32, (_C, _C), 0)
    col = jax.lax.broadcasted_iota(jnp.int32, (_C, _C), 1)
    lmat = jnp.where(col <= row, 1.0, 0.0).astype(jnp.float32)
    mask_ge = col >= row

    def chunk(c, carry):
        t0 = c * _C
        new_carry = list(carry)
        for bb in range(_BG):
            xc = x_ref[bb, pl.ds(t0, _C), :]
            z = jnp.dot(xc, wgT[...],
                        preferred_element_type=jnp.float32)
            sig = jax.nn.sigmoid(z)
            lsp = _logsig(z)
            lsn = _logsig(-z)
            ll = jnp.concatenate([lsp[:, _H:2 * _H], lsn[:, 2 * _H:3 * _H]],
                                 axis=1)
            cum = jnp.dot(lmat, ll, precision=_HI,
                          preferred_element_type=jnp.float32)
            tot = cum[_C - 1:_C, :]
            le = cum[:, :_H]
            la = tot[:, :_H] - le
            lb = tot[:, _H:] - cum[:, _H:]
            lbe = lb + le
            a_row = jnp.exp(tot[:, :_H])
            ca_row = jnp.exp(tot[:, _H:])
            sme_row = jnp.sum(jnp.exp(lbe), axis=0, keepdims=True)
            ela = jnp.exp(la)
            lbe_t = lbe.T

            for h in range(_H):
                p = bb * _H + h
                p1t, p2t, m1t, m2t = new_carry[p]
                fsl = slice(h * _HD, (h + 1) * _HD)
                kc = ks[bb, pl.ds(t0, _C), fsl]
                vc = vs[bb, pl.ds(t0, _C), fsl]
                qc = qs[bb, pl.ds(t0, _C), fsl]
                kn = jnp.sqrt(jnp.sum(kc * kc, axis=1, keepdims=True))
                kc = kc / jnp.maximum(kn, 1e-12)
                qn = jnp.sqrt(jnp.sum(qc * qc, axis=1, keepdims=True))
                qc = qc / jnp.maximum(qn, 1e-12)

                qk = jnp.concatenate([qc, kc], axis=0)
                pre2 = jnp.dot(qk, p1t,
                               preferred_element_type=jnp.float32)
                act = _silu(pre2)
                mid2 = jnp.dot(act, p2t,
                               preferred_element_type=jnp.float32)
                outq = qc + mid2[:_C, :]
                rs[p, pl.ds(t0, _C), :] = outq

                pre = pre2[_C:, :]
                h1 = act[_C:, :]
                y = kc + mid2[_C:, :]
                e = 2.0 * (y - vc)
                dsv = jax.lax.dot_general(
                    e, p2t, (((1,), (1,)), ((), ())),
                    preferred_element_type=jnp.float32) * _dsilu(pre)

                e2 = jnp.sum(e * e, axis=1, keepdims=True)
                h12 = jnp.sum(h1 * h1, axis=1, keepdims=True)
                d2 = jnp.sum(dsv * dsv, axis=1, keepdims=True)
                k2 = jnp.sum(kc * kc, axis=1, keepdims=True)
                s2 = 1.0 / jnp.maximum(jnp.sqrt(e2 * h12) / _MAXN, 1.0)
                s1 = 1.0 / jnp.maximum(jnp.sqrt(d2 * k2) / _MAXN, 1.0)

                diff = lbe_t[h:h + 1, :] - le[:, h:h + 1]
                cmat = jnp.exp(jnp.where(mask_ge, diff, -1e30))
                ccol = jnp.sum(cmat, axis=1, keepdims=True)

                lr_col = sig[:, h:h + 1]
                wm1 = ela[:, h:h + 1] * lr_col * s1
                wp1 = ccol * lr_col * s1
                wm2 = ela[:, h:h + 1] * lr_col * s2
                wp2 = ccol * lr_col * s2

                dn_ta = (((0,), (0,)), ((), ()))
                dup1 = _dot3(
                    _sp(kc),
                    _sp(jnp.concatenate([dsv * wm1, dsv * wp1], axis=1)),
                    dn_ta)
                dup2 = _dot3(
                    _sp(h1),
                    _sp(jnp.concatenate([e * wm2, e * wp2], axis=1)),
                    dn_ta)
                dm1, dp1 = dup1[:, :_HID], dup1[:, _HID:]
                dm2, dp2 = dup2[:, :_HD], dup2[:, _HD:]

                a_s = a_row[0:1, h:h + 1]
                ca_s = ca_row[0:1, h:h + 1]
                sm_s = sme_row[0:1, h:h + 1]
                new_carry[p] = (ca_s * p1t + sm_s * m1t - dp1,
                                ca_s * p2t + sm_s * m2t - dp2,
                                a_s * m1t - dm1,
                                a_s * m2t - dm2)
        return tuple(new_carry)

    init = tuple((w1T[h], w2T[h],
                  jnp.zeros((_HD, _HID), jnp.float32),
                  jnp.zeros((_HID, _HD), jnp.float32))
                 for bb in range(_BG) for h in range(_H))

    def chunk2(i, carry):
        return chunk(2 * i + 1, chunk(2 * i, carry))

    jax.lax.fori_loop(0, _NC // 2, chunk2, init)

    for bb in range(_BG):
        for r in range(4):
            sl = slice(r * 256, (r + 1) * 256)
            acc = jnp.dot(rs[bb * _H, sl, :], woutT[0:_HD, :],
                          preferred_element_type=jnp.float32)
            for h in range(1, _H):
                acc = acc + jnp.dot(rs[bb * _H + h, sl, :],
                                    woutT[h * _HD:(h + 1) * _HD, :],
                                    preferred_element_type=jnp.float32)
            out_ref[bb, sl, :] = acc


def _call(x, wkT, wvT, wqT, wgT, w1T, w2T, woutT, interpret=False):
    return pl.pallas_call(
        _body,
        out_shape=jax.ShapeDtypeStruct((_B, _S, _D), jnp.float32),
        grid=(_B // _BG,),
        in_specs=[
            pl.BlockSpec((_BG, _S, _D), lambda i: (i, 0, 0)),
            pl.BlockSpec((_D, _D), lambda i: (0, 0)),
            pl.BlockSpec((_D, _D), lambda i: (0, 0)),
            pl.BlockSpec((_D, _D), lambda i: (0, 0)),
            pl.BlockSpec((_D, 3 * _H), lambda i: (0, 0)),
            pl.BlockSpec((_H, _HD, _HID), lambda i: (0, 0, 0)),
            pl.BlockSpec((_H, _HID, _HD), lambda i: (0, 0, 0)),
            pl.BlockSpec((_D, _D), lambda i: (0, 0)),
        ],
        out_specs=pl.BlockSpec((_BG, _S, _D), lambda i: (i, 0, 0)),
        scratch_shapes=[
            pltpu.VMEM((_BG, _S, _D), jnp.float32),
            pltpu.VMEM((_BG, _S, _D), jnp.float32),
            pltpu.VMEM((_BG, _S, _D), jnp.float32),
            pltpu.VMEM((_G, _S, _HD), jnp.float32),
        ],
        compiler_params=pltpu.CompilerParams(
            dimension_semantics=("parallel",),
        ),
        name="neural_memory",
        interpret=interpret,
    )(x, wkT, wvT, wqT, wgT, w1T, w2T, woutT)


def kernel(x, Wk, Wv, Wq, Wlr, Wmom, Wdec, W1, W2, Wout, interpret=False):
    wgT = jnp.concatenate([Wlr, Wmom, Wdec], axis=0).T
    return _call(x, Wk.T, Wv.T, Wq.T, wgT,
                 jnp.swapaxes(W1, 1, 2), jnp.swapaxes(W2, 1, 2), Wout.T,
                 interpret=interpret)


# speedup vs baseline: 1.2358x; 1.0016x over previous
import jax
import jax.numpy as jnp
from jax.experimental import pallas as pl
from jax.experimental.pallas import tpu as pltpu

_B, _S, _D = 4, 1024, 256
_H, _HD = 4, 64
_HID = 128
_C = 64
_NC = _S // _C
_MAXN = 10.0
_BG = 2
_G = _BG * _H

_HI = jax.lax.Precision.HIGHEST


def _sp(x):
    hi = x.astype(jnp.bfloat16)
    lo = (x - hi.astype(jnp.float32)).astype(jnp.bfloat16)
    return hi, lo


def _dot3(a, b, dn=None):
    ah, al = a
    bh, bl = b
    if dn is None:
        f = lambda u, v: jnp.dot(u, v, preferred_element_type=jnp.float32)
    else:
        f = lambda u, v: jax.lax.dot_general(
            u, v, dn, preferred_element_type=jnp.float32)
    return f(ah, bh) + (f(ah, bl) + f(al, bh))


def _logsig(z):
    return -(jnp.maximum(-z, 0.0) + jnp.log(1.0 + jnp.exp(-jnp.abs(z))))


def _silu(z):
    return z * jax.nn.sigmoid(z)


def _dsilu(z):
    s = jax.nn.sigmoid(z)
    return s * (1.0 + z * (1.0 - s))


def _body(x_ref, wkT, wvT, wqT, wgT, w1T, w2T, woutT, out_ref, ks, vs, qs, rs):
    wk = wkT[...]
    wv = wvT[...]
    wq = wqT[...]
    for bb in range(_BG):
        for r in range(4):
            sl = slice(r * 256, (r + 1) * 256)
            xb = x_ref[bb, sl, :]
            ks[bb, sl, :] = jnp.dot(xb, wk, preferred_element_type=jnp.float32)
            vs[bb, sl, :] = jnp.dot(xb, wv, preferred_element_type=jnp.float32)
            qs[bb, sl, :] = jnp.dot(xb, wq, preferred_element_type=jnp.float32)

    row = jax.lax.broadcasted_iota(jnp.int32, (_C, _C), 0)
    col = jax.lax.broadcasted_iota(jnp.int32, (_C, _C), 1)
    lmat = jnp.where(col <= row, 1.0, 0.0).astype(jnp.float32)
    mask_ge = col >= row

    def chunk(c, carry):
        t0 = c * _C
        new_carry = list(carry)
        for bb in range(_BG):
            xc = x_ref[bb, pl.ds(t0, _C), :]
            z = jnp.dot(xc, wgT[...],
                        preferred_element_type=jnp.float32)
            sig = jax.nn.sigmoid(z)
            lsp = _logsig(z)
            lsn = _logsig(-z)
            ll = jnp.concatenate([lsp[:, _H:2 * _H], lsn[:, 2 * _H:3 * _H]],
                                 axis=1)
            cum = jnp.dot(lmat, ll, precision=_HI,
                          preferred_element_type=jnp.float32)
            tot = cum[_C - 1:_C, :]
            le = cum[:, :_H]
            la = tot[:, :_H] - le
            lb = tot[:, _H:] - cum[:, _H:]
            lbe = lb + le
            a_row = jnp.exp(tot[:, :_H])
            ca_row = jnp.exp(tot[:, _H:])
            sme_row = jnp.sum(jnp.exp(lbe), axis=0, keepdims=True)
            ela = jnp.exp(la)
            lbe_t = lbe.T

            for h in range(_H):
                p = bb * _H + h
                p1t, p2t, m1t, m2t = new_carry[p]
                fsl = slice(h * _HD, (h + 1) * _HD)
                kc = ks[bb, pl.ds(t0, _C), fsl]
                vc = vs[bb, pl.ds(t0, _C), fsl]
                qc = qs[bb, pl.ds(t0, _C), fsl]
                kn = jnp.sqrt(jnp.sum(kc * kc, axis=1, keepdims=True))
                kc = kc / jnp.maximum(kn, 1e-12)
                qn = jnp.sqrt(jnp.sum(qc * qc, axis=1, keepdims=True))
                qc = qc / jnp.maximum(qn, 1e-12)

                qk = jnp.concatenate([qc, kc], axis=0)
                pre2 = jnp.dot(qk, p1t,
                               preferred_element_type=jnp.float32)
                act = _silu(pre2)
                mid2 = jnp.dot(act, p2t,
                               preferred_element_type=jnp.float32)
                outq = qc + mid2[:_C, :]
                rs[p, pl.ds(t0, _C), :] = outq

                pre = pre2[_C:, :]
                h1 = act[_C:, :]
                y = kc + mid2[_C:, :]
                e = 2.0 * (y - vc)
                dsv = jax.lax.dot_general(
                    e, p2t, (((1,), (1,)), ((), ())),
                    preferred_element_type=jnp.float32) * _dsilu(pre)

                e2 = jnp.sum(e * e, axis=1, keepdims=True)
                h12 = jnp.sum(h1 * h1, axis=1, keepdims=True)
                d2 = jnp.sum(dsv * dsv, axis=1, keepdims=True)
                k2 = jnp.sum(kc * kc, axis=1, keepdims=True)
                s2 = 1.0 / jnp.maximum(jnp.sqrt(e2 * h12) / _MAXN, 1.0)
                s1 = 1.0 / jnp.maximum(jnp.sqrt(d2 * k2) / _MAXN, 1.0)

                diff = lbe_t[h:h + 1, :] - le[:, h:h + 1]
                cmat = jnp.exp(jnp.where(mask_ge, diff, -1e30))
                ccol = jnp.sum(cmat, axis=1, keepdims=True)

                lr_col = sig[:, h:h + 1]
                wm1 = ela[:, h:h + 1] * lr_col * s1
                wp1 = ccol * lr_col * s1
                wm2 = ela[:, h:h + 1] * lr_col * s2
                wp2 = ccol * lr_col * s2

                dn_ta = (((0,), (0,)), ((), ()))
                dup1 = _dot3(
                    _sp(kc),
                    _sp(jnp.concatenate([dsv * wm1, dsv * wp1], axis=1)),
                    dn_ta)
                dup2 = _dot3(
                    _sp(h1),
                    _sp(jnp.concatenate([e * wm2, e * wp2], axis=1)),
                    dn_ta)
                dm1, dp1 = dup1[:, :_HID], dup1[:, _HID:]
                dm2, dp2 = dup2[:, :_HD], dup2[:, _HD:]

                a_s = a_row[0:1, h:h + 1]
                ca_s = ca_row[0:1, h:h + 1]
                sm_s = sme_row[0:1, h:h + 1]
                new_carry[p] = (ca_s * p1t + sm_s * m1t - dp1,
                                ca_s * p2t + sm_s * m2t - dp2,
                                a_s * m1t - dm1,
                                a_s * m2t - dm2)
        return tuple(new_carry)

    init = tuple((w1T[h], w2T[h],
                  jnp.zeros((_HD, _HID), jnp.float32),
                  jnp.zeros((_HID, _HD), jnp.float32))
                 for bb in range(_BG) for h in range(_H))

    def chunk4(i, carry):
        for j in range(4):
            carry = chunk(4 * i + j, carry)
        return carry

    jax.lax.fori_loop(0, _NC // 4, chunk4, init)

    for bb in range(_BG):
        for r in range(4):
            sl = slice(r * 256, (r + 1) * 256)
            acc = jnp.dot(rs[bb * _H, sl, :], woutT[0:_HD, :],
                          preferred_element_type=jnp.float32)
            for h in range(1, _H):
                acc = acc + jnp.dot(rs[bb * _H + h, sl, :],
                                    woutT[h * _HD:(h + 1) * _HD, :],
                                    preferred_element_type=jnp.float32)
            out_ref[bb, sl, :] = acc


def _call(x, wkT, wvT, wqT, wgT, w1T, w2T, woutT, interpret=False):
    return pl.pallas_call(
        _body,
        out_shape=jax.ShapeDtypeStruct((_B, _S, _D), jnp.float32),
        grid=(_B // _BG,),
        in_specs=[
            pl.BlockSpec((_BG, _S, _D), lambda i: (i, 0, 0)),
            pl.BlockSpec((_D, _D), lambda i: (0, 0)),
            pl.BlockSpec((_D, _D), lambda i: (0, 0)),
            pl.BlockSpec((_D, _D), lambda i: (0, 0)),
            pl.BlockSpec((_D, 3 * _H), lambda i: (0, 0)),
            pl.BlockSpec((_H, _HD, _HID), lambda i: (0, 0, 0)),
            pl.BlockSpec((_H, _HID, _HD), lambda i: (0, 0, 0)),
            pl.BlockSpec((_D, _D), lambda i: (0, 0)),
        ],
        out_specs=pl.BlockSpec((_BG, _S, _D), lambda i: (i, 0, 0)),
        scratch_shapes=[
            pltpu.VMEM((_BG, _S, _D), jnp.float32),
            pltpu.VMEM((_BG, _S, _D), jnp.float32),
            pltpu.VMEM((_BG, _S, _D), jnp.float32),
            pltpu.VMEM((_G, _S, _HD), jnp.float32),
        ],
        compiler_params=pltpu.CompilerParams(
            dimension_semantics=("parallel",),
        ),
        name="neural_memory",
        interpret=interpret,
    )(x, wkT, wvT, wqT, wgT, w1T, w2T, woutT)


def kernel(x, Wk, Wv, Wq, Wlr, Wmom, Wdec, W1, W2, Wout, interpret=False):
    wgT = jnp.concatenate([Wlr, Wmom, Wdec], axis=0).T
    return _call(x, Wk.T, Wv.T, Wq.T, wgT,
                 jnp.swapaxes(W1, 1, 2), jnp.swapaxes(W2, 1, 2), Wout.T,
                 interpret=interpret)


# speedup vs baseline: 1.2370x; 1.0010x over previous
import jax
import jax.numpy as jnp
from jax.experimental import pallas as pl
from jax.experimental.pallas import tpu as pltpu

_B, _S, _D = 4, 1024, 256
_H, _HD = 4, 64
_HID = 128
_C = 64
_NC = _S // _C
_MAXN = 10.0
_BG = 2
_G = _BG * _H

_HI = jax.lax.Precision.HIGHEST


def _sp(x):
    hi = x.astype(jnp.bfloat16)
    lo = (x - hi.astype(jnp.float32)).astype(jnp.bfloat16)
    return hi, lo


def _dot3(a, b, dn=None):
    ah, al = a
    bh, bl = b
    if dn is None:
        f = lambda u, v: jnp.dot(u, v, preferred_element_type=jnp.float32)
    else:
        f = lambda u, v: jax.lax.dot_general(
            u, v, dn, preferred_element_type=jnp.float32)
    return f(ah, bh) + (f(ah, bl) + f(al, bh))


def _logsig(z):
    return -(jnp.maximum(-z, 0.0) + jnp.log(1.0 + jnp.exp(-jnp.abs(z))))


def _silu(z):
    return z * jax.nn.sigmoid(z)


def _dsilu(z):
    s = jax.nn.sigmoid(z)
    return s * (1.0 + z * (1.0 - s))


def _body(x_ref, wkT, wvT, wqT, wgT, w1T, w2T, woutT, out_ref, ks, vs, qs, rs):
    wk = wkT[...]
    wv = wvT[...]
    wq = wqT[...]
    for bb in range(_BG):
        for r in range(4):
            sl = slice(r * 256, (r + 1) * 256)
            xb = x_ref[bb, sl, :]
            ks[bb, sl, :] = jnp.dot(xb, wk, preferred_element_type=jnp.float32)
            vs[bb, sl, :] = jnp.dot(xb, wv, preferred_element_type=jnp.float32)
            qs[bb, sl, :] = jnp.dot(xb, wq, preferred_element_type=jnp.float32)

    row = jax.lax.broadcasted_iota(jnp.int32, (_C, _C), 0)
    col = jax.lax.broadcasted_iota(jnp.int32, (_C, _C), 1)
    lmat = jnp.where(col <= row, 1.0, 0.0).astype(jnp.float32)
    mask_ge = col >= row

    def chunk(c, carry):
        t0 = c * _C
        new_carry = list(carry)
        for bb in range(_BG):
            xc = x_ref[bb, pl.ds(t0, _C), :]
            z = jnp.dot(xc, wgT[...],
                        preferred_element_type=jnp.float32)
            sig = jax.nn.sigmoid(z)
            lsp = _logsig(z)
            lsn = _logsig(-z)
            ll = jnp.concatenate([lsp[:, _H:2 * _H], lsn[:, 2 * _H:3 * _H]],
                                 axis=1)
            cum = jnp.dot(lmat, ll, precision=_HI,
                          preferred_element_type=jnp.float32)
            tot = cum[_C - 1:_C, :]
            le = cum[:, :_H]
            la = tot[:, :_H] - le
            lb = tot[:, _H:] - cum[:, _H:]
            lbe = lb + le
            a_row = jnp.exp(tot[:, :_H])
            ca_row = jnp.exp(tot[:, _H:])
            sme_row = jnp.sum(jnp.exp(lbe), axis=0, keepdims=True)
            ela = jnp.exp(la)
            lbe_t = lbe.T

            for h in range(_H):
                p = bb * _H + h
                p1t, p2t, m1t, m2t = new_carry[p]
                fsl = slice(h * _HD, (h + 1) * _HD)
                kc = ks[bb, pl.ds(t0, _C), fsl]
                vc = vs[bb, pl.ds(t0, _C), fsl]
                qc = qs[bb, pl.ds(t0, _C), fsl]
                kn = jnp.sqrt(jnp.sum(kc * kc, axis=1, keepdims=True))
                kc = kc / jnp.maximum(kn, 1e-12)
                qn = jnp.sqrt(jnp.sum(qc * qc, axis=1, keepdims=True))
                qc = qc / jnp.maximum(qn, 1e-12)

                qk = jnp.concatenate([qc, kc], axis=0)
                pre2 = jnp.dot(qk, p1t,
                               preferred_element_type=jnp.float32)
                act = _silu(pre2)
                mid2 = jnp.dot(act, p2t,
                               preferred_element_type=jnp.float32)
                outq = qc + mid2[:_C, :]
                rs[p, pl.ds(t0, _C), :] = outq

                pre = pre2[_C:, :]
                h1 = act[_C:, :]
                y = kc + mid2[_C:, :]
                e = 2.0 * (y - vc)
                dsv = jax.lax.dot_general(
                    e, p2t, (((1,), (1,)), ((), ())),
                    preferred_element_type=jnp.float32) * _dsilu(pre)

                e2 = jnp.sum(e * e, axis=1, keepdims=True)
                h12 = jnp.sum(h1 * h1, axis=1, keepdims=True)
                d2 = jnp.sum(dsv * dsv, axis=1, keepdims=True)
                k2 = jnp.sum(kc * kc, axis=1, keepdims=True)
                s2 = 1.0 / jnp.maximum(jnp.sqrt(e2 * h12) / _MAXN, 1.0)
                s1 = 1.0 / jnp.maximum(jnp.sqrt(d2 * k2) / _MAXN, 1.0)

                diff = lbe_t[h:h + 1, :] - le[:, h:h + 1]
                cmat = jnp.exp(jnp.where(mask_ge, diff, -1e30))
                ccol = jnp.sum(cmat, axis=1, keepdims=True)

                lr_col = sig[:, h:h + 1]
                wm1 = ela[:, h:h + 1] * lr_col * s1
                wp1 = ccol * lr_col * s1
                wm2 = ela[:, h:h + 1] * lr_col * s2
                wp2 = ccol * lr_col * s2

                dn_ta = (((0,), (0,)), ((), ()))
                dup1 = _dot3(
                    _sp(kc),
                    _sp(jnp.concatenate([dsv * wm1, dsv * wp1], axis=1)),
                    dn_ta)
                dup2 = _dot3(
                    _sp(h1),
                    _sp(jnp.concatenate([e * wm2, e * wp2], axis=1)),
                    dn_ta)
                dm1, dp1 = dup1[:, :_HID], dup1[:, _HID:]
                dm2, dp2 = dup2[:, :_HD], dup2[:, _HD:]

                a_s = a_row[0:1, h:h + 1]
                ca_s = ca_row[0:1, h:h + 1]
                sm_s = sme_row[0:1, h:h + 1]
                new_carry[p] = (ca_s * p1t + sm_s * m1t - dp1,
                                ca_s * p2t + sm_s * m2t - dp2,
                                a_s * m1t - dm1,
                                a_s * m2t - dm2)
        return tuple(new_carry)

    init = tuple((w1T[h], w2T[h],
                  jnp.zeros((_HD, _HID), jnp.float32),
                  jnp.zeros((_HID, _HD), jnp.float32))
                 for bb in range(_BG) for h in range(_H))

    def chunk4(i, carry):
        for j in range(4):
            carry = chunk(4 * i + j, carry)
        return carry

    jax.lax.fori_loop(0, _NC // 4, chunk4, init)

    for bb in range(_BG):
        for r in range(4):
            sl = slice(r * 256, (r + 1) * 256)
            acc = jnp.dot(rs[bb * _H, sl, :], woutT[0:_HD, :],
                          preferred_element_type=jnp.float32)
            for h in range(1, _H):
                acc = acc + jnp.dot(rs[bb * _H + h, sl, :],
                                    woutT[h * _HD:(h + 1) * _HD, :],
                                    preferred_element_type=jnp.float32)
            out_ref[bb, sl, :] = acc


def _call(x, wkT, wvT, wqT, wgT, w1T, w2T, woutT):
    return pl.pallas_call(
        _body,
        out_shape=jax.ShapeDtypeStruct((_B, _S, _D), jnp.float32),
        grid=(_B // _BG,),
        in_specs=[
            pl.BlockSpec((_BG, _S, _D), lambda i: (i, 0, 0)),
            pl.BlockSpec((_D, _D), lambda i: (0, 0)),
            pl.BlockSpec((_D, _D), lambda i: (0, 0)),
            pl.BlockSpec((_D, _D), lambda i: (0, 0)),
            pl.BlockSpec((_D, 3 * _H), lambda i: (0, 0)),
            pl.BlockSpec((_H, _HD, _HID), lambda i: (0, 0, 0)),
            pl.BlockSpec((_H, _HID, _HD), lambda i: (0, 0, 0)),
            pl.BlockSpec((_D, _D), lambda i: (0, 0)),
        ],
        out_specs=pl.BlockSpec((_BG, _S, _D), lambda i: (i, 0, 0)),
        scratch_shapes=[
            pltpu.VMEM((_BG, _S, _D), jnp.float32),
            pltpu.VMEM((_BG, _S, _D), jnp.float32),
            pltpu.VMEM((_BG, _S, _D), jnp.float32),
            pltpu.VMEM((_G, _S, _HD), jnp.float32),
        ],
        compiler_params=pltpu.CompilerParams(
            dimension_semantics=("parallel",),
        ),
        name="neural_memory",
    )(x, wkT, wvT, wqT, wgT, w1T, w2T, woutT)


def kernel(x, Wk, Wv, Wq, Wlr, Wmom, Wdec, W1, W2, Wout):
    wgT = jnp.concatenate([Wlr, Wmom, Wdec], axis=0).T
    return _call(x, Wk.T, Wv.T, Wq.T, wgT,
                 jnp.swapaxes(W1, 1, 2), jnp.swapaxes(W2, 1, 2), Wout.T)
